# baseline (speedup 1.0000x reference)
import jax
import jax.numpy as jnp
from jax.experimental import pallas as pl
from jax.experimental.pallas import tpu as pltpu
from jax.experimental.pallas import tpu_sc as plsc

N = 50000
D_IN = 1024
D1 = 512
D2 = 256
K_SAMPLE = 8
BLK = 2000
PAD_ROWS = 392
NEG_INF = float("-inf")
POS_INF = float("inf")


def _dot(x, y):
    yb = y if y.dtype == jnp.bfloat16 else y.astype(jnp.bfloat16)
    return jax.lax.dot_general(
        x.astype(jnp.bfloat16), yb,
        (((x.ndim - 1,), (0,)), ((), ())),
        preferred_element_type=jnp.float32)


def _stream_body(h_ref, w1_ref, b1_ref, wab_ref, bab_ref,
                 wc_ref, bc_ref, att_ref, wsum_ref, ssum_ref):
    i = pl.program_id(0)

    @pl.when(i == 0)
    def _():
        wsum_ref[...] = jnp.zeros_like(wsum_ref)
        ssum_ref[...] = jnp.zeros_like(ssum_ref)

    feat = jnp.maximum(_dot(h_ref[...], w1_ref[...]) + b1_ref[...], 0.0)
    ab = _dot(feat, wab_ref[...]) + bab_ref[...]
    a = jnp.tanh(ab[:, :D2])
    b = jax.nn.sigmoid(ab[:, D2:])
    att = _dot(a * b, wc_ref[...]) + bc_ref[0, 0]
    att_ref[...] = att
    sig = jax.nn.sigmoid(att)
    wsum_ref[...] += jnp.sum(feat * sig, axis=0, keepdims=True)
    ssum_ref[...] += jnp.sum(sig, axis=0, keepdims=True)


def _stream(h, W1, b1, Wab, bab, Wc, bc):
    return pl.pallas_call(
        _stream_body,
        grid=(N // BLK,),
        in_specs=[
            pl.BlockSpec((BLK, D_IN), lambda i: (i, 0)),
            pl.BlockSpec((D_IN, D1), lambda i: (0, 0)),
            pl.BlockSpec((1, D1), lambda i: (0, 0)),
            pl.BlockSpec((D1, 2 * D2), lambda i: (0, 0)),
            pl.BlockSpec((1, 2 * D2), lambda i: (0, 0)),
            pl.BlockSpec((D2, 1), lambda i: (0, 0)),
            pl.BlockSpec(memory_space=pltpu.SMEM),
        ],
        out_specs=[
            pl.BlockSpec((BLK, 1), lambda i: (i, 0)),
            pl.BlockSpec((1, D1), lambda i: (0, 0)),
            pl.BlockSpec((1, 1), lambda i: (0, 0)),
        ],
        out_shape=[
            jax.ShapeDtypeStruct((N, 1), jnp.float32),
            jax.ShapeDtypeStruct((1, D1), jnp.float32),
            jax.ShapeDtypeStruct((1, 1), jnp.float32),
        ],
        compiler_params=pltpu.CompilerParams(
            dimension_semantics=("arbitrary",)),
    )(h, W1, b1, Wab, bab, Wc, bc)


def _tree(parts, op):
    while len(parts) > 1:
        nxt = [op(parts[j], parts[j + 1])
               for j in range(0, len(parts) - 1, 2)]
        if len(parts) % 2:
            nxt.append(parts[-1])
        parts = nxt
    return parts[0]


def _split(x):
    return [x[k * 8:(k + 1) * 8] for k in range(x.shape[0] // 8)]


def _topk_body(att_ref, ids_ref):
    vals = att_ref[...]
    row = jax.lax.broadcasted_iota(jnp.int32, (PAD_ROWS, 128), 0)
    col = jax.lax.broadcasted_iota(jnp.int32, (PAD_ROWS, 128), 1)
    lin = row * 128 + col
    valid = lin < N
    big = jnp.int32(2**31 - 1)
    vt = jnp.where(valid, vals, NEG_INF)
    vb = jnp.where(valid, vals, POS_INF)
    for k in range(K_SAMPLE):
        m = jnp.max(_tree(_split(vt), jnp.maximum))
        idx = jnp.min(_tree(_split(jnp.where(vt == m, lin, big)),
                            jnp.minimum))
        ids_ref[0, k] = idx
        vt = jnp.where(lin == idx, NEG_INF, vt)
    for k in range(K_SAMPLE):
        m = jnp.min(_tree(_split(vb), jnp.minimum))
        idx = jnp.min(_tree(_split(jnp.where(vb == m, lin, big)),
                            jnp.minimum))
        ids_ref[0, K_SAMPLE + k] = idx
        vb = jnp.where(lin == idx, POS_INF, vb)


def _topk(att_pad):
    return pl.pallas_call(
        _topk_body,
        in_specs=[pl.BlockSpec((PAD_ROWS, 128), lambda: (0, 0))],
        out_specs=pl.BlockSpec(memory_space=pltpu.SMEM),
        out_shape=jax.ShapeDtypeStruct((1, 2 * K_SAMPLE), jnp.int32),
    )(att_pad)


def _gather_rows(h, ids):
    mesh = plsc.VectorSubcoreMesh(core_axis_name="c", subcore_axis_name="s")

    @pl.kernel(out_type=jax.ShapeDtypeStruct((2 * K_SAMPLE, D_IN),
                                             jnp.float32),
               mesh=mesh,
               scratch_types=[pltpu.VMEM((1, 2 * K_SAMPLE), jnp.int32),
                              pltpu.VMEM((2 * K_SAMPLE, D_IN), jnp.float32),
                              pltpu.SemaphoreType.DMA])
    def kern(h_hbm, ids_hbm, o_hbm, ids_vmem, buf, sem):
        c = jax.lax.axis_index("c")
        s = jax.lax.axis_index("s")

        @pl.when(jnp.logical_and(c == 0, s == 0))
        def _():
            pltpu.async_copy(ids_hbm, ids_vmem, sem).wait()
            pltpu.sync_copy(h_hbm.at[ids_vmem.at[0]], buf)
            pltpu.async_copy(buf, o_hbm, sem).wait()

    return kern(h, ids)


def _tail_body(hg_ref, w1_ref, b1_ref, wi0_ref, bi0_ref, wi1_ref, bi1_ref,
               wcls_ref, bcls_ref, wsum_ref, ssum_ref, lab_ref, iev_ref,
               logits_ref, prob_ref, yhat_ref, loss_ref):
    fg = jnp.maximum(_dot(hg_ref[...], w1_ref[...]) + b1_ref[...], 0.0)

    def ce(lg):
        m = jnp.max(lg, axis=1, keepdims=True)
        lse = m + jnp.log(jnp.sum(jnp.exp(lg - m), axis=1, keepdims=True))
        rid = jax.lax.broadcasted_iota(jnp.int32, (2 * K_SAMPLE, 1), 0)
        ll = jnp.where(rid < K_SAMPLE, lg[:, 1:2], lg[:, 0:1])
        return jnp.sum(lse - ll) / (2.0 * K_SAMPLE)

    l0 = ce(_dot(fg, wi0_ref[...]) + bi0_ref[...])
    l1 = ce(_dot(fg, wi1_ref[...]) + bi1_ref[...])
    lab = lab_ref[0, 0]
    iev = iev_ref[0, 0]
    loss_ref[0, 0] = jnp.where(
        iev != 0, jnp.where(lab == 0, l0, l1), jnp.float32(0.0))

    bag = wsum_ref[...] / ssum_ref[0, 0]
    lg = _dot(bag, wcls_ref[...]) + bcls_ref[...]
    logits_ref[...] = lg
    mm = jnp.max(lg, axis=1, keepdims=True)
    e = jnp.exp(lg - mm)
    prob_ref[...] = e / jnp.sum(e, axis=1, keepdims=True)
    yhat_ref[0, 0] = jnp.where(lg[0, 1] > lg[0, 0], 1, 0).astype(jnp.int32)


def _tail(hg, W1, b1, Wi0, bi0, Wi1, bi1, Wcls, bcls, wsum, ssum, lab, iev):
    vm = lambda shape: pl.BlockSpec(shape, lambda: tuple(0 for _ in shape))
    sm = pl.BlockSpec(memory_space=pltpu.SMEM)
    return pl.pallas_call(
        _tail_body,
        in_specs=[
            vm((2 * K_SAMPLE, D_IN)), vm((D_IN, D1)), vm((1, D1)),
            vm((D1, 2)), vm((1, 2)), vm((D1, 2)), vm((1, 2)),
            vm((D1, 2)), vm((1, 2)), vm((1, D1)), sm, sm, sm,
        ],
        out_specs=[vm((1, 2)), vm((1, 2)), sm, sm],
        out_shape=[
            jax.ShapeDtypeStruct((1, 2), jnp.float32),
            jax.ShapeDtypeStruct((1, 2), jnp.float32),
            jax.ShapeDtypeStruct((1, 1), jnp.int32),
            jax.ShapeDtypeStruct((1, 1), jnp.float32),
        ],
    )(hg, W1, b1, Wi0, bi0, Wi1, bi1, Wcls, bcls, wsum, ssum, lab, iev)


def kernel(h, label, instance_eval, W1, b1, Wa, ba, Wb, bb, Wc, bc,
           Wcls, bcls, Wi0, bi0, Wi1, bi1):
    W1b = W1.astype(jnp.bfloat16)
    Wab = jnp.concatenate([Wa, Wb], axis=1).astype(jnp.bfloat16)
    bab = jnp.concatenate([ba, bb]).reshape(1, 2 * D2)
    att2d, wsum, ssum = _stream(
        h, W1b, b1.reshape(1, D1), Wab, bab,
        Wc.astype(jnp.bfloat16), bc.reshape(1, 1))

    att_pad = jnp.pad(att2d.reshape(-1),
                      (0, PAD_ROWS * 128 - N)).reshape(PAD_ROWS, 128)
    ids = _topk(att_pad)

    hg = _gather_rows(h, ids)

    lab = label.reshape(1, 1).astype(jnp.int32)
    iev = jnp.asarray(instance_eval, jnp.int32).reshape(1, 1)
    logits, prob, yhat, loss = _tail(
        hg, W1b, b1.reshape(1, D1), Wi0, bi0.reshape(1, 2),
        Wi1, bi1.reshape(1, 2), Wcls, bcls.reshape(1, 2),
        wsum, ssum, lab, iev)

    return (logits, prob, yhat, att2d.reshape(1, N), loss.reshape(()))


# speedup vs baseline: 1.0386x; 1.0386x over previous
import jax
import jax.numpy as jnp
from jax.experimental import pallas as pl
from jax.experimental.pallas import tpu as pltpu
from jax.experimental.pallas import tpu_sc as plsc

N = 50000
D_IN = 1024
D1 = 512
D2 = 256
K_SAMPLE = 8
BLK = 2000
PAD_ROWS = 392
NEG_INF = float("-inf")
POS_INF = float("inf")


def _dot(x, y):
    xb = x if x.dtype == jnp.bfloat16 else x.astype(jnp.bfloat16)
    yb = y if y.dtype == jnp.bfloat16 else y.astype(jnp.bfloat16)
    return jax.lax.dot_general(
        xb, yb, (((x.ndim - 1,), (0,)), ((), ())),
        preferred_element_type=jnp.float32)


def _stream_body(h_ref, w1_ref, b1_ref, wa_ref, ba_ref, wb_ref, bb_ref,
                 wc_ref, bc_ref, att_ref, wsum_ref, ssum_ref):
    i = pl.program_id(0)

    @pl.when(i == 0)
    def _():
        wsum_ref[...] = jnp.zeros_like(wsum_ref)
        ssum_ref[...] = jnp.zeros_like(ssum_ref)

    feat = jnp.maximum(_dot(h_ref[...], w1_ref[...]) + b1_ref[...],
                       0.0).astype(jnp.bfloat16)
    a = jnp.tanh(_dot(feat, wa_ref[...]) + ba_ref[...])
    b = jax.nn.sigmoid(_dot(feat, wb_ref[...]) + bb_ref[...])
    att = _dot(a * b, wc_ref[...]) + bc_ref[0, 0]
    att_ref[...] = att
    sig = jax.nn.sigmoid(att)
    wsum_ref[...] += jnp.sum(feat.astype(jnp.float32) * sig,
                             axis=0, keepdims=True)
    ssum_ref[...] += jnp.sum(sig, axis=0, keepdims=True)


def _stream(h, W1, b1, Wa, ba, Wb, bb, Wc, bc):
    return pl.pallas_call(
        _stream_body,
        grid=(N // BLK,),
        in_specs=[
            pl.BlockSpec((BLK, D_IN), lambda i: (i, 0)),
            pl.BlockSpec((D_IN, D1), lambda i: (0, 0)),
            pl.BlockSpec((1, D1), lambda i: (0, 0)),
            pl.BlockSpec((D1, D2), lambda i: (0, 0)),
            pl.BlockSpec((1, D2), lambda i: (0, 0)),
            pl.BlockSpec((D1, D2), lambda i: (0, 0)),
            pl.BlockSpec((1, D2), lambda i: (0, 0)),
            pl.BlockSpec((D2, 1), lambda i: (0, 0)),
            pl.BlockSpec(memory_space=pltpu.SMEM),
        ],
        out_specs=[
            pl.BlockSpec((BLK, 1), lambda i: (i, 0)),
            pl.BlockSpec((1, D1), lambda i: (0, 0)),
            pl.BlockSpec((1, 1), lambda i: (0, 0)),
        ],
        out_shape=[
            jax.ShapeDtypeStruct((N, 1), jnp.float32),
            jax.ShapeDtypeStruct((1, D1), jnp.float32),
            jax.ShapeDtypeStruct((1, 1), jnp.float32),
        ],
        compiler_params=pltpu.CompilerParams(
            dimension_semantics=("arbitrary",)),
    )(h, W1, b1, Wa, ba, Wb, bb, Wc, bc)


def _tree(parts, op):
    while len(parts) > 1:
        nxt = [op(parts[j], parts[j + 1])
               for j in range(0, len(parts) - 1, 2)]
        if len(parts) % 2:
            nxt.append(parts[-1])
        parts = nxt
    return parts[0]


def _split(x):
    return [x[k * 8:(k + 1) * 8] for k in range(x.shape[0] // 8)]


def _topk_body(att_ref, ids_ref):
    vals = att_ref[...]
    row = jax.lax.broadcasted_iota(jnp.int32, (PAD_ROWS, 128), 0)
    col = jax.lax.broadcasted_iota(jnp.int32, (PAD_ROWS, 128), 1)
    lin = row * 128 + col
    valid = lin < N
    big = jnp.int32(2**31 - 1)
    vt = jnp.where(valid, vals, NEG_INF)
    vb = jnp.where(valid, vals, POS_INF)
    for k in range(K_SAMPLE):
        m = jnp.max(_tree(_split(vt), jnp.maximum))
        idx = jnp.min(_tree(_split(jnp.where(vt == m, lin, big)),
                            jnp.minimum))
        ids_ref[0, k] = idx
        vt = jnp.where(lin == idx, NEG_INF, vt)
    for k in range(K_SAMPLE):
        m = jnp.min(_tree(_split(vb), jnp.minimum))
        idx = jnp.min(_tree(_split(jnp.where(vb == m, lin, big)),
                            jnp.minimum))
        ids_ref[0, K_SAMPLE + k] = idx
        vb = jnp.where(lin == idx, POS_INF, vb)


def _topk(att_pad):
    return pl.pallas_call(
        _topk_body,
        in_specs=[pl.BlockSpec((PAD_ROWS, 128), lambda: (0, 0))],
        out_specs=pl.BlockSpec(memory_space=pltpu.SMEM),
        out_shape=jax.ShapeDtypeStruct((1, 2 * K_SAMPLE), jnp.int32),
    )(att_pad)


def _gather_rows(h, ids):
    mesh = plsc.VectorSubcoreMesh(core_axis_name="c", subcore_axis_name="s")

    @pl.kernel(out_type=jax.ShapeDtypeStruct((2 * K_SAMPLE, D_IN),
                                             jnp.float32),
               mesh=mesh,
               scratch_types=[pltpu.VMEM((1, 2 * K_SAMPLE), jnp.int32),
                              pltpu.VMEM((2 * K_SAMPLE, D_IN), jnp.float32),
                              pltpu.SemaphoreType.DMA])
    def kern(h_hbm, ids_hbm, o_hbm, ids_vmem, buf, sem):
        c = jax.lax.axis_index("c")
        s = jax.lax.axis_index("s")

        @pl.when(jnp.logical_and(c == 0, s == 0))
        def _():
            pltpu.async_copy(ids_hbm, ids_vmem, sem).wait()
            pltpu.sync_copy(h_hbm.at[ids_vmem.at[0]], buf)
            pltpu.async_copy(buf, o_hbm, sem).wait()

    return kern(h, ids)


def _tail_body(hg_ref, w1_ref, b1_ref, wi0_ref, bi0_ref, wi1_ref, bi1_ref,
               wcls_ref, bcls_ref, wsum_ref, ssum_ref, lab_ref, iev_ref,
               logits_ref, prob_ref, yhat_ref, loss_ref):
    fg = jnp.maximum(_dot(hg_ref[...], w1_ref[...]) + b1_ref[...], 0.0)

    def ce(lg):
        m = jnp.max(lg, axis=1, keepdims=True)
        lse = m + jnp.log(jnp.sum(jnp.exp(lg - m), axis=1, keepdims=True))
        rid = jax.lax.broadcasted_iota(jnp.int32, (2 * K_SAMPLE, 1), 0)
        ll = jnp.where(rid < K_SAMPLE, lg[:, 1:2], lg[:, 0:1])
        return jnp.sum(lse - ll) / (2.0 * K_SAMPLE)

    l0 = ce(_dot(fg, wi0_ref[...]) + bi0_ref[...])
    l1 = ce(_dot(fg, wi1_ref[...]) + bi1_ref[...])
    lab = lab_ref[0, 0]
    iev = iev_ref[0, 0]
    loss_ref[0, 0] = jnp.where(
        iev != 0, jnp.where(lab == 0, l0, l1), jnp.float32(0.0))

    bag = wsum_ref[...] / ssum_ref[0, 0]
    lg = _dot(bag, wcls_ref[...]) + bcls_ref[...]
    logits_ref[...] = lg
    mm = jnp.max(lg, axis=1, keepdims=True)
    e = jnp.exp(lg - mm)
    prob_ref[...] = e / jnp.sum(e, axis=1, keepdims=True)
    yhat_ref[0, 0] = jnp.where(lg[0, 1] > lg[0, 0], 1, 0).astype(jnp.int32)


def _tail(hg, W1, b1, Wi0, bi0, Wi1, bi1, Wcls, bcls, wsum, ssum, lab, iev):
    vm = lambda shape: pl.BlockSpec(shape, lambda: tuple(0 for _ in shape))
    sm = pl.BlockSpec(memory_space=pltpu.SMEM)
    return pl.pallas_call(
        _tail_body,
        in_specs=[
            vm((2 * K_SAMPLE, D_IN)), vm((D_IN, D1)), vm((1, D1)),
            vm((D1, 2)), vm((1, 2)), vm((D1, 2)), vm((1, 2)),
            vm((D1, 2)), vm((1, 2)), vm((1, D1)), sm, sm, sm,
        ],
        out_specs=[vm((1, 2)), vm((1, 2)), sm, sm],
        out_shape=[
            jax.ShapeDtypeStruct((1, 2), jnp.float32),
            jax.ShapeDtypeStruct((1, 2), jnp.float32),
            jax.ShapeDtypeStruct((1, 1), jnp.int32),
            jax.ShapeDtypeStruct((1, 1), jnp.float32),
        ],
    )(hg, W1, b1, Wi0, bi0, Wi1, bi1, Wcls, bcls, wsum, ssum, lab, iev)


def kernel(h, label, instance_eval, W1, b1, Wa, ba, Wb, bb, Wc, bc,
           Wcls, bcls, Wi0, bi0, Wi1, bi1):
    att2d, wsum, ssum = _stream(
        h, W1, b1.reshape(1, D1), Wa, ba.reshape(1, D2),
        Wb, bb.reshape(1, D2), Wc, bc.reshape(1, 1))

    att_pad = jnp.pad(att2d.reshape(-1),
                      (0, PAD_ROWS * 128 - N)).reshape(PAD_ROWS, 128)
    ids = _topk(att_pad)

    hg = _gather_rows(h, ids)

    lab = label.reshape(1, 1).astype(jnp.int32)
    iev = jnp.asarray(instance_eval, jnp.int32).reshape(1, 1)
    logits, prob, yhat, loss = _tail(
        hg, W1, b1.reshape(1, D1), Wi0, bi0.reshape(1, 2),
        Wi1, bi1.reshape(1, 2), Wcls, bcls.reshape(1, 2),
        wsum, ssum, lab, iev)

    return (logits, prob, yhat, att2d.reshape(1, N), loss.reshape(()))


# speedup vs baseline: 1.1335x; 1.0914x over previous
import jax
import jax.numpy as jnp
from jax.experimental import pallas as pl
from jax.experimental.pallas import tpu as pltpu
from jax.experimental.pallas import tpu_sc as plsc

N = 50000
D_IN = 1024
D1 = 512
D2 = 256
K_SAMPLE = 8
BLK = 2048
NBLK = -(-N // BLK)
ATT_W = NBLK * BLK
PAD_ROWS = ATT_W // 128
NEG_INF = float("-inf")
POS_INF = float("inf")


def _dot(x, y):
    xb = x if x.dtype == jnp.bfloat16 else x.astype(jnp.bfloat16)
    yb = y if y.dtype == jnp.bfloat16 else y.astype(jnp.bfloat16)
    return jax.lax.dot_general(
        xb, yb, (((x.ndim - 1,), (0,)), ((), ())),
        preferred_element_type=jnp.float32)


def _stream_body(h_ref, w1_ref, b1_ref, wa_ref, ba_ref, wb_ref, bb_ref,
                 wc_ref, bc_ref, att_ref, wsum_ref, ssum_ref):
    i = pl.program_id(0)

    @pl.when(i == 0)
    def _():
        wsum_ref[...] = jnp.zeros_like(wsum_ref)
        ssum_ref[...] = jnp.zeros_like(ssum_ref)

    rowv = i * BLK + jax.lax.broadcasted_iota(jnp.int32, (BLK, 1), 0) < N
    feat = jnp.where(rowv,
                     jnp.maximum(_dot(h_ref[...], w1_ref[...]) + b1_ref[...],
                                 0.0), 0.0).astype(jnp.bfloat16)
    a = jnp.tanh(_dot(feat, wa_ref[...]) + ba_ref[...])
    b = jax.nn.sigmoid(_dot(feat, wb_ref[...]) + bb_ref[...])
    g = (a * b).astype(jnp.bfloat16)
    att = jax.lax.dot_general(
        wc_ref[...].astype(jnp.bfloat16), g, (((0,), (1,)), ((), ())),
        preferred_element_type=jnp.float32) + bc_ref[0, 0]
    att_ref[...] = att
    lane = jax.lax.broadcasted_iota(jnp.int32, (1, BLK), 1)
    valid = i * BLK + lane < N
    sig = jnp.where(valid, jax.nn.sigmoid(att), 0.0)
    wsum_ref[...] += jax.lax.dot_general(
        sig.astype(jnp.bfloat16), feat, (((1,), (0,)), ((), ())),
        preferred_element_type=jnp.float32)
    ssum_ref[...] += jnp.sum(sig, axis=1, keepdims=True)


def _stream(h, W1, b1, Wa, ba, Wb, bb, Wc, bc):
    return pl.pallas_call(
        _stream_body,
        grid=(NBLK,),
        in_specs=[
            pl.BlockSpec((BLK, D_IN), lambda i: (i, 0)),
            pl.BlockSpec((D_IN, D1), lambda i: (0, 0)),
            pl.BlockSpec((1, D1), lambda i: (0, 0)),
            pl.BlockSpec((D1, D2), lambda i: (0, 0)),
            pl.BlockSpec((1, D2), lambda i: (0, 0)),
            pl.BlockSpec((D1, D2), lambda i: (0, 0)),
            pl.BlockSpec((1, D2), lambda i: (0, 0)),
            pl.BlockSpec((D2, 1), lambda i: (0, 0)),
            pl.BlockSpec(memory_space=pltpu.SMEM),
        ],
        out_specs=[
            pl.BlockSpec((1, BLK), lambda i: (0, i)),
            pl.BlockSpec((1, D1), lambda i: (0, 0)),
            pl.BlockSpec((1, 1), lambda i: (0, 0)),
        ],
        out_shape=[
            jax.ShapeDtypeStruct((1, ATT_W), jnp.float32),
            jax.ShapeDtypeStruct((1, D1), jnp.float32),
            jax.ShapeDtypeStruct((1, 1), jnp.float32),
        ],
        compiler_params=pltpu.CompilerParams(
            dimension_semantics=("arbitrary",)),
    )(h, W1, b1, Wa, ba, Wb, bb, Wc, bc)


def _tree(parts, op):
    while len(parts) > 1:
        nxt = [op(parts[j], parts[j + 1])
               for j in range(0, len(parts) - 1, 2)]
        if len(parts) % 2:
            nxt.append(parts[-1])
        parts = nxt
    return parts[0]


def _split(x):
    return [x[k * 8:(k + 1) * 8] for k in range(x.shape[0] // 8)]


def _topk_body(att_ref, ids_ref):
    vals = att_ref[...]
    row = jax.lax.broadcasted_iota(jnp.int32, (PAD_ROWS, 128), 0)
    col = jax.lax.broadcasted_iota(jnp.int32, (PAD_ROWS, 128), 1)
    lin = row * 128 + col
    valid = lin < N
    big = jnp.int32(2**31 - 1)
    vt = jnp.where(valid, vals, NEG_INF)
    vb = jnp.where(valid, vals, POS_INF)
    for k in range(K_SAMPLE):
        m = jnp.max(_tree(_split(vt), jnp.maximum))
        idx = jnp.min(_tree(_split(jnp.where(vt == m, lin, big)),
                            jnp.minimum))
        ids_ref[0, k] = idx
        vt = jnp.where(lin == idx, NEG_INF, vt)
    for k in range(K_SAMPLE):
        m = jnp.min(_tree(_split(vb), jnp.minimum))
        idx = jnp.min(_tree(_split(jnp.where(vb == m, lin, big)),
                            jnp.minimum))
        ids_ref[0, K_SAMPLE + k] = idx
        vb = jnp.where(lin == idx, POS_INF, vb)


def _topk(att_pad):
    return pl.pallas_call(
        _topk_body,
        in_specs=[pl.BlockSpec((PAD_ROWS, 128), lambda: (0, 0))],
        out_specs=pl.BlockSpec(memory_space=pltpu.SMEM),
        out_shape=jax.ShapeDtypeStruct((1, 2 * K_SAMPLE), jnp.int32),
    )(att_pad)


def _gather_rows(h, ids):
    mesh = plsc.VectorSubcoreMesh(core_axis_name="c", subcore_axis_name="s")

    @pl.kernel(out_type=jax.ShapeDtypeStruct((2 * K_SAMPLE, D_IN),
                                             jnp.float32),
               mesh=mesh,
               scratch_types=[pltpu.VMEM((1, 2 * K_SAMPLE), jnp.int32),
                              pltpu.VMEM((2 * K_SAMPLE, D_IN), jnp.float32),
                              pltpu.SemaphoreType.DMA])
    def kern(h_hbm, ids_hbm, o_hbm, ids_vmem, buf, sem):
        c = jax.lax.axis_index("c")
        s = jax.lax.axis_index("s")

        @pl.when(jnp.logical_and(c == 0, s == 0))
        def _():
            pltpu.async_copy(ids_hbm, ids_vmem, sem).wait()
            pltpu.sync_copy(h_hbm.at[ids_vmem.at[0]], buf)
            pltpu.async_copy(buf, o_hbm, sem).wait()

    return kern(h, ids)


def _tail_body(hg_ref, w1_ref, b1_ref, wi0_ref, bi0_ref, wi1_ref, bi1_ref,
               wcls_ref, bcls_ref, wsum_ref, ssum_ref, lab_ref, iev_ref,
               logits_ref, prob_ref, yhat_ref, loss_ref):
    fg = jnp.maximum(_dot(hg_ref[...], w1_ref[...]) + b1_ref[...], 0.0)

    def ce(lg):
        m = jnp.max(lg, axis=1, keepdims=True)
        lse = m + jnp.log(jnp.sum(jnp.exp(lg - m), axis=1, keepdims=True))
        rid = jax.lax.broadcasted_iota(jnp.int32, (2 * K_SAMPLE, 1), 0)
        ll = jnp.where(rid < K_SAMPLE, lg[:, 1:2], lg[:, 0:1])
        return jnp.sum(lse - ll) / (2.0 * K_SAMPLE)

    l0 = ce(_dot(fg, wi0_ref[...]) + bi0_ref[...])
    l1 = ce(_dot(fg, wi1_ref[...]) + bi1_ref[...])
    lab = lab_ref[0, 0]
    iev = iev_ref[0, 0]
    loss_ref[0, 0] = jnp.where(
        iev != 0, jnp.where(lab == 0, l0, l1), jnp.float32(0.0))

    bag = wsum_ref[...] / ssum_ref[0, 0]
    lg = _dot(bag, wcls_ref[...]) + bcls_ref[...]
    logits_ref[...] = lg
    mm = jnp.max(lg, axis=1, keepdims=True)
    e = jnp.exp(lg - mm)
    prob_ref[...] = e / jnp.sum(e, axis=1, keepdims=True)
    yhat_ref[0, 0] = jnp.where(lg[0, 1] > lg[0, 0], 1, 0).astype(jnp.int32)


def _tail(hg, W1, b1, Wi0, bi0, Wi1, bi1, Wcls, bcls, wsum, ssum, lab, iev):
    vm = lambda shape: pl.BlockSpec(shape, lambda: tuple(0 for _ in shape))
    sm = pl.BlockSpec(memory_space=pltpu.SMEM)
    return pl.pallas_call(
        _tail_body,
        in_specs=[
            vm((2 * K_SAMPLE, D_IN)), vm((D_IN, D1)), vm((1, D1)),
            vm((D1, 2)), vm((1, 2)), vm((D1, 2)), vm((1, 2)),
            vm((D1, 2)), vm((1, 2)), vm((1, D1)), sm, sm, sm,
        ],
        out_specs=[vm((1, 2)), vm((1, 2)), sm, sm],
        out_shape=[
            jax.ShapeDtypeStruct((1, 2), jnp.float32),
            jax.ShapeDtypeStruct((1, 2), jnp.float32),
            jax.ShapeDtypeStruct((1, 1), jnp.int32),
            jax.ShapeDtypeStruct((1, 1), jnp.float32),
        ],
    )(hg, W1, b1, Wi0, bi0, Wi1, bi1, Wcls, bcls, wsum, ssum, lab, iev)


def kernel(h, label, instance_eval, W1, b1, Wa, ba, Wb, bb, Wc, bc,
           Wcls, bcls, Wi0, bi0, Wi1, bi1):
    att_full, wsum, ssum = _stream(
        h, W1, b1.reshape(1, D1), Wa, ba.reshape(1, D2),
        Wb, bb.reshape(1, D2), Wc, bc.reshape(1, 1))

    ids = _topk(att_full.reshape(PAD_ROWS, 128))

    hg = _gather_rows(h, ids)

    lab = label.reshape(1, 1).astype(jnp.int32)
    iev = jnp.asarray(instance_eval, jnp.int32).reshape(1, 1)
    logits, prob, yhat, loss = _tail(
        hg, W1, b1.reshape(1, D1), Wi0, bi0.reshape(1, 2),
        Wi1, bi1.reshape(1, 2), Wcls, bcls.reshape(1, 2),
        wsum, ssum, lab, iev)

    return (logits, prob, yhat, att_full[:, :N], loss.reshape(()))


# speedup vs baseline: 1.1355x; 1.0017x over previous
import jax
import jax.numpy as jnp
from jax.experimental import pallas as pl
from jax.experimental.pallas import tpu as pltpu
from jax.experimental.pallas import tpu_sc as plsc

N = 50000
D_IN = 1024
D1 = 512
D2 = 256
K_SAMPLE = 8
BLK = 2048
NBLK = -(-N // BLK)
ATT_W = NBLK * BLK
PAD_ROWS = ATT_W // 128
NEG_INF = float("-inf")
POS_INF = float("inf")


def _dot(x, y):
    xb = x if x.dtype == jnp.bfloat16 else x.astype(jnp.bfloat16)
    yb = y if y.dtype == jnp.bfloat16 else y.astype(jnp.bfloat16)
    return jax.lax.dot_general(
        xb, yb, (((x.ndim - 1,), (0,)), ((), ())),
        preferred_element_type=jnp.float32)


def _stream_body(h_ref, w1_ref, b1_ref, wa_ref, ba_ref, wb_ref, bb_ref,
                 wc_ref, bc_ref, att_ref, att2_ref, wsum_ref, ssum_ref):
    i = pl.program_id(0)

    @pl.when(i == 0)
    def _():
        wsum_ref[...] = jnp.zeros_like(wsum_ref)
        ssum_ref[...] = jnp.zeros_like(ssum_ref)

    rowv = i * BLK + jax.lax.broadcasted_iota(jnp.int32, (BLK, 1), 0) < N
    feat = jnp.where(rowv,
                     jnp.maximum(_dot(h_ref[...], w1_ref[...]) + b1_ref[...],
                                 0.0), 0.0).astype(jnp.bfloat16)
    a = jnp.tanh(_dot(feat, wa_ref[...]) + ba_ref[...])
    b = jax.nn.sigmoid(_dot(feat, wb_ref[...]) + bb_ref[...])
    g = (a * b).astype(jnp.bfloat16)
    att = jax.lax.dot_general(
        wc_ref[...].astype(jnp.bfloat16), g, (((0,), (1,)), ((), ())),
        preferred_element_type=jnp.float32) + bc_ref[0, 0]
    att_ref[...] = att
    att2_ref[...] = att.reshape(BLK // 128, 128)
    lane = jax.lax.broadcasted_iota(jnp.int32, (1, BLK), 1)
    valid = i * BLK + lane < N
    sig = jnp.where(valid, jax.nn.sigmoid(att), 0.0)
    wsum_ref[...] += jax.lax.dot_general(
        sig.astype(jnp.bfloat16), feat, (((1,), (0,)), ((), ())),
        preferred_element_type=jnp.float32)
    ssum_ref[...] += jnp.sum(sig, axis=1, keepdims=True)


def _stream(h, W1, b1, Wa, ba, Wb, bb, Wc, bc):
    return pl.pallas_call(
        _stream_body,
        grid=(NBLK,),
        in_specs=[
            pl.BlockSpec((BLK, D_IN), lambda i: (i, 0)),
            pl.BlockSpec((D_IN, D1), lambda i: (0, 0)),
            pl.BlockSpec((1, D1), lambda i: (0, 0)),
            pl.BlockSpec((D1, D2), lambda i: (0, 0)),
            pl.BlockSpec((1, D2), lambda i: (0, 0)),
            pl.BlockSpec((D1, D2), lambda i: (0, 0)),
            pl.BlockSpec((1, D2), lambda i: (0, 0)),
            pl.BlockSpec((D2, 1), lambda i: (0, 0)),
            pl.BlockSpec(memory_space=pltpu.SMEM),
        ],
        out_specs=[
            pl.BlockSpec((1, BLK), lambda i: (0, i)),
            pl.BlockSpec((BLK // 128, 128), lambda i: (i, 0)),
            pl.BlockSpec((1, D1), lambda i: (0, 0)),
            pl.BlockSpec((1, 1), lambda i: (0, 0)),
        ],
        out_shape=[
            jax.ShapeDtypeStruct((1, N), jnp.float32),
            jax.ShapeDtypeStruct((PAD_ROWS, 128), jnp.float32),
            jax.ShapeDtypeStruct((1, D1), jnp.float32),
            jax.ShapeDtypeStruct((1, 1), jnp.float32),
        ],
        compiler_params=pltpu.CompilerParams(
            dimension_semantics=("arbitrary",)),
    )(h, W1, b1, Wa, ba, Wb, bb, Wc, bc)


def _tree(parts, op):
    while len(parts) > 1:
        nxt = [op(parts[j], parts[j + 1])
               for j in range(0, len(parts) - 1, 2)]
        if len(parts) % 2:
            nxt.append(parts[-1])
        parts = nxt
    return parts[0]


def _split(x):
    return [x[k * 8:(k + 1) * 8] for k in range(x.shape[0] // 8)]


def _topk_body(att_ref, ids_ref):
    vals = att_ref[...]
    row = jax.lax.broadcasted_iota(jnp.int32, (PAD_ROWS, 128), 0)
    col = jax.lax.broadcasted_iota(jnp.int32, (PAD_ROWS, 128), 1)
    lin = row * 128 + col
    valid = lin < N
    big = jnp.int32(2**31 - 1)
    vt = jnp.where(valid, vals, NEG_INF)
    vb = jnp.where(valid, vals, POS_INF)
    for k in range(K_SAMPLE):
        m = jnp.max(_tree(_split(vt), jnp.maximum))
        idx = jnp.min(_tree(_split(jnp.where(vt == m, lin, big)),
                            jnp.minimum))
        ids_ref[0, k] = idx
        vt = jnp.where(lin == idx, NEG_INF, vt)
    for k in range(K_SAMPLE):
        m = jnp.min(_tree(_split(vb), jnp.minimum))
        idx = jnp.min(_tree(_split(jnp.where(vb == m, lin, big)),
                            jnp.minimum))
        ids_ref[0, K_SAMPLE + k] = idx
        vb = jnp.where(lin == idx, POS_INF, vb)


def _topk(att_pad):
    return pl.pallas_call(
        _topk_body,
        in_specs=[pl.BlockSpec((PAD_ROWS, 128), lambda: (0, 0))],
        out_specs=pl.BlockSpec(memory_space=pltpu.SMEM),
        out_shape=jax.ShapeDtypeStruct((1, 2 * K_SAMPLE), jnp.int32),
    )(att_pad)


def _gather_rows(h, ids):
    mesh = plsc.VectorSubcoreMesh(core_axis_name="c", subcore_axis_name="s")

    @pl.kernel(out_type=jax.ShapeDtypeStruct((2 * K_SAMPLE, D_IN),
                                             jnp.float32),
               mesh=mesh,
               scratch_types=[pltpu.VMEM((1, 2 * K_SAMPLE), jnp.int32),
                              pltpu.VMEM((2 * K_SAMPLE, D_IN), jnp.float32),
                              pltpu.SemaphoreType.DMA])
    def kern(h_hbm, ids_hbm, o_hbm, ids_vmem, buf, sem):
        c = jax.lax.axis_index("c")
        s = jax.lax.axis_index("s")

        @pl.when(jnp.logical_and(c == 0, s == 0))
        def _():
            pltpu.async_copy(ids_hbm, ids_vmem, sem).wait()
            pltpu.sync_copy(h_hbm.at[ids_vmem.at[0]], buf)
            pltpu.async_copy(buf, o_hbm, sem).wait()

    return kern(h, ids)


def _tail_body(hg_ref, w1_ref, b1_ref, wi0_ref, bi0_ref, wi1_ref, bi1_ref,
               wcls_ref, bcls_ref, wsum_ref, ssum_ref, lab_ref, iev_ref,
               logits_ref, prob_ref, yhat_ref, loss_ref):
    fg = jnp.maximum(_dot(hg_ref[...], w1_ref[...]) + b1_ref[...], 0.0)

    def ce(lg):
        m = jnp.max(lg, axis=1, keepdims=True)
        lse = m + jnp.log(jnp.sum(jnp.exp(lg - m), axis=1, keepdims=True))
        rid = jax.lax.broadcasted_iota(jnp.int32, (2 * K_SAMPLE, 1), 0)
        ll = jnp.where(rid < K_SAMPLE, lg[:, 1:2], lg[:, 0:1])
        return jnp.sum(lse - ll) / (2.0 * K_SAMPLE)

    l0 = ce(_dot(fg, wi0_ref[...]) + bi0_ref[...])
    l1 = ce(_dot(fg, wi1_ref[...]) + bi1_ref[...])
    lab = lab_ref[0, 0]
    iev = iev_ref[0, 0]
    loss_ref[0, 0] = jnp.where(
        iev != 0, jnp.where(lab == 0, l0, l1), jnp.float32(0.0))

    bag = wsum_ref[...] / ssum_ref[0, 0]
    lg = _dot(bag, wcls_ref[...]) + bcls_ref[...]
    logits_ref[...] = lg
    mm = jnp.max(lg, axis=1, keepdims=True)
    e = jnp.exp(lg - mm)
    prob_ref[...] = e / jnp.sum(e, axis=1, keepdims=True)
    yhat_ref[0, 0] = jnp.where(lg[0, 1] > lg[0, 0], 1, 0).astype(jnp.int32)


def _tail(hg, W1, b1, Wi0, bi0, Wi1, bi1, Wcls, bcls, wsum, ssum, lab, iev):
    vm = lambda shape: pl.BlockSpec(shape, lambda: tuple(0 for _ in shape))
    sm = pl.BlockSpec(memory_space=pltpu.SMEM)
    return pl.pallas_call(
        _tail_body,
        in_specs=[
            vm((2 * K_SAMPLE, D_IN)), vm((D_IN, D1)), vm((1, D1)),
            vm((D1, 2)), vm((1, 2)), vm((D1, 2)), vm((1, 2)),
            vm((D1, 2)), vm((1, 2)), vm((1, D1)), sm, sm, sm,
        ],
        out_specs=[vm((1, 2)), vm((1, 2)), sm, sm],
        out_shape=[
            jax.ShapeDtypeStruct((1, 2), jnp.float32),
            jax.ShapeDtypeStruct((1, 2), jnp.float32),
            jax.ShapeDtypeStruct((1, 1), jnp.int32),
            jax.ShapeDtypeStruct((1, 1), jnp.float32),
        ],
    )(hg, W1, b1, Wi0, bi0, Wi1, bi1, Wcls, bcls, wsum, ssum, lab, iev)


def kernel(h, label, instance_eval, W1, b1, Wa, ba, Wb, bb, Wc, bc,
           Wcls, bcls, Wi0, bi0, Wi1, bi1):
    att_row, att_pad, wsum, ssum = _stream(
        h, W1, b1.reshape(1, D1), Wa, ba.reshape(1, D2),
        Wb, bb.reshape(1, D2), Wc, bc.reshape(1, 1))

    ids = _topk(att_pad)

    hg = _gather_rows(h, ids)

    lab = label.reshape(1, 1).astype(jnp.int32)
    iev = jnp.asarray(instance_eval, jnp.int32).reshape(1, 1)
    logits, prob, yhat, loss = _tail(
        hg, W1, b1.reshape(1, D1), Wi0, bi0.reshape(1, 2),
        Wi1, bi1.reshape(1, 2), Wcls, bcls.reshape(1, 2),
        wsum, ssum, lab, iev)

    return (logits, prob, yhat, att_row, loss.reshape(()))


# speedup vs baseline: 1.1478x; 1.0108x over previous
import jax
import jax.numpy as jnp
from jax.experimental import pallas as pl
from jax.experimental.pallas import tpu as pltpu
from jax.experimental.pallas import tpu_sc as plsc

N = 50000
D_IN = 1024
D1 = 512
D2 = 256
K_SAMPLE = 8
BLK = 2048
NBLK = -(-N // BLK)
ATT_W = NBLK * BLK
PAD_ROWS = ATT_W // 128
NEG_INF = float("-inf")
POS_INF = float("inf")


def _dot(x, y):
    xb = x if x.dtype == jnp.bfloat16 else x.astype(jnp.bfloat16)
    yb = y if y.dtype == jnp.bfloat16 else y.astype(jnp.bfloat16)
    return jax.lax.dot_general(
        xb, yb, (((x.ndim - 1,), (0,)), ((), ())),
        preferred_element_type=jnp.float32)


def _stream_body(h_ref, w1_ref, b1_ref, wa_ref, ba_ref, wb_ref, bb_ref,
                 wc_ref, bc_ref, att_ref, att2_ref, wsum_ref, ssum_ref):
    i = pl.program_id(0)

    @pl.when(i == 0)
    def _():
        wsum_ref[...] = jnp.zeros_like(wsum_ref)
        ssum_ref[...] = jnp.zeros_like(ssum_ref)

    rowv = i * BLK + jax.lax.broadcasted_iota(jnp.int32, (BLK, 1), 0) < N
    feat = jnp.where(rowv,
                     jnp.maximum(_dot(h_ref[...], w1_ref[...]) + b1_ref[...],
                                 0.0), 0.0).astype(jnp.bfloat16)
    a = jnp.tanh(_dot(feat, wa_ref[...]) + ba_ref[...])
    b = jax.nn.sigmoid(_dot(feat, wb_ref[...]) + bb_ref[...])
    g = (a * b).astype(jnp.bfloat16)
    att = jax.lax.dot_general(
        wc_ref[...].astype(jnp.bfloat16), g, (((0,), (1,)), ((), ())),
        preferred_element_type=jnp.float32) + bc_ref[0, 0]
    att_ref[...] = att
    att2_ref[...] = att.reshape(BLK // 128, 128)
    lane = jax.lax.broadcasted_iota(jnp.int32, (1, BLK), 1)
    valid = i * BLK + lane < N
    sig = jnp.where(valid, jax.nn.sigmoid(att), 0.0)
    wsum_ref[...] += jax.lax.dot_general(
        sig.astype(jnp.bfloat16), feat, (((1,), (0,)), ((), ())),
        preferred_element_type=jnp.float32)
    ssum_ref[...] += jnp.sum(sig, axis=1, keepdims=True)


def _stream(h, W1, b1, Wa, ba, Wb, bb, Wc, bc):
    return pl.pallas_call(
        _stream_body,
        grid=(NBLK,),
        in_specs=[
            pl.BlockSpec((BLK, D_IN), lambda i: (i, 0)),
            pl.BlockSpec((D_IN, D1), lambda i: (0, 0)),
            pl.BlockSpec((1, D1), lambda i: (0, 0)),
            pl.BlockSpec((D1, D2), lambda i: (0, 0)),
            pl.BlockSpec((1, D2), lambda i: (0, 0)),
            pl.BlockSpec((D1, D2), lambda i: (0, 0)),
            pl.BlockSpec((1, D2), lambda i: (0, 0)),
            pl.BlockSpec((D2, 1), lambda i: (0, 0)),
            pl.BlockSpec(memory_space=pltpu.SMEM),
        ],
        out_specs=[
            pl.BlockSpec((1, BLK), lambda i: (0, i)),
            pl.BlockSpec((BLK // 128, 128), lambda i: (i, 0)),
            pl.BlockSpec((1, D1), lambda i: (0, 0)),
            pl.BlockSpec((1, 1), lambda i: (0, 0)),
        ],
        out_shape=[
            jax.ShapeDtypeStruct((1, N), jnp.float32),
            jax.ShapeDtypeStruct((PAD_ROWS, 128), jnp.float32),
            jax.ShapeDtypeStruct((1, D1), jnp.float32),
            jax.ShapeDtypeStruct((1, 1), jnp.float32),
        ],
        compiler_params=pltpu.CompilerParams(
            dimension_semantics=("arbitrary",)),
    )(h, W1, b1, Wa, ba, Wb, bb, Wc, bc)


def _tree(parts, op):
    while len(parts) > 1:
        nxt = [op(parts[j], parts[j + 1])
               for j in range(0, len(parts) - 1, 2)]
        if len(parts) % 2:
            nxt.append(parts[-1])
        parts = nxt
    return parts[0]


def _split(x):
    return [x[k * 8:(k + 1) * 8] for k in range(x.shape[0] // 8)]


def _colwise_top8(vals, rowi, lin, maximize):
    sentinel = NEG_INF if maximize else POS_INF
    red = jnp.maximum if maximize else jnp.minimum
    bigrow = jnp.int32(2**31 - 1)
    v = vals
    out_v, out_l = [], []
    for _ in range(K_SAMPLE):
        m = _tree(_split(v), red)
        m = _tree([m[j:j + 1] for j in range(8)], red)
        r = _tree(_split(jnp.where(v == m, rowi, bigrow)), jnp.minimum)
        r = _tree([r[j:j + 1] for j in range(8)], jnp.minimum)
        out_v.append(m)
        out_l.append(r * 128 + jax.lax.broadcasted_iota(
            jnp.int32, (1, 128), 1))
        v = jnp.where(rowi == r, sentinel, v)
    return jnp.concatenate(out_v, 0), jnp.concatenate(out_l, 0)


def _topk_body(att_ref, ids_ref):
    vals = att_ref[...]
    row = jax.lax.broadcasted_iota(jnp.int32, (PAD_ROWS, 128), 0)
    col = jax.lax.broadcasted_iota(jnp.int32, (PAD_ROWS, 128), 1)
    lin = row * 128 + col
    valid = lin < N
    big = jnp.int32(2**31 - 1)
    vt = jnp.where(valid, vals, NEG_INF)
    vb = jnp.where(valid, vals, POS_INF)
    tv, tl = _colwise_top8(vt, row, lin, True)
    bv, bl = _colwise_top8(vb, row, lin, False)
    for k in range(K_SAMPLE):
        mt = jnp.max(tv)
        it = jnp.min(jnp.where(tv == mt, tl, big))
        ids_ref[0, k] = it
        tv = jnp.where(tl == it, NEG_INF, tv)
        mb = jnp.min(bv)
        ib = jnp.min(jnp.where(bv == mb, bl, big))
        ids_ref[0, K_SAMPLE + k] = ib
        bv = jnp.where(bl == ib, POS_INF, bv)


def _topk(att_pad):
    return pl.pallas_call(
        _topk_body,
        in_specs=[pl.BlockSpec((PAD_ROWS, 128), lambda: (0, 0))],
        out_specs=pl.BlockSpec(memory_space=pltpu.SMEM),
        out_shape=jax.ShapeDtypeStruct((1, 2 * K_SAMPLE), jnp.int32),
    )(att_pad)


def _gather_rows(h, ids):
    mesh = plsc.VectorSubcoreMesh(core_axis_name="c", subcore_axis_name="s")

    @pl.kernel(out_type=jax.ShapeDtypeStruct((2 * K_SAMPLE, D_IN),
                                             jnp.float32),
               mesh=mesh,
               scratch_types=[pltpu.VMEM((1, 2 * K_SAMPLE), jnp.int32),
                              pltpu.VMEM((1, D_IN), jnp.float32),
                              pltpu.SemaphoreType.DMA])
    def kern(h_hbm, ids_hbm, o_hbm, ids_vmem, buf, sem):
        c = jax.lax.axis_index("c")
        s = jax.lax.axis_index("s")

        @pl.when(c == 0)
        def _():
            pltpu.async_copy(ids_hbm, ids_vmem, sem).wait()
            pltpu.sync_copy(h_hbm.at[ids_vmem.at[0, pl.ds(s, 1)]], buf)
            pltpu.async_copy(buf, o_hbm.at[pl.ds(s, 1)], sem).wait()

    return kern(h, ids)


def _tail_body(hg_ref, w1_ref, b1_ref, wi0_ref, bi0_ref, wi1_ref, bi1_ref,
               wcls_ref, bcls_ref, wsum_ref, ssum_ref, lab_ref, iev_ref,
               logits_ref, prob_ref, yhat_ref, loss_ref):
    fg = jnp.maximum(_dot(hg_ref[...], w1_ref[...]) + b1_ref[...], 0.0)

    def ce(lg):
        m = jnp.max(lg, axis=1, keepdims=True)
        lse = m + jnp.log(jnp.sum(jnp.exp(lg - m), axis=1, keepdims=True))
        rid = jax.lax.broadcasted_iota(jnp.int32, (2 * K_SAMPLE, 1), 0)
        ll = jnp.where(rid < K_SAMPLE, lg[:, 1:2], lg[:, 0:1])
        return jnp.sum(lse - ll) / (2.0 * K_SAMPLE)

    l0 = ce(_dot(fg, wi0_ref[...]) + bi0_ref[...])
    l1 = ce(_dot(fg, wi1_ref[...]) + bi1_ref[...])
    lab = lab_ref[0, 0]
    iev = iev_ref[0, 0]
    loss_ref[0, 0] = jnp.where(
        iev != 0, jnp.where(lab == 0, l0, l1), jnp.float32(0.0))

    bag = wsum_ref[...] / ssum_ref[0, 0]
    lg = _dot(bag, wcls_ref[...]) + bcls_ref[...]
    logits_ref[...] = lg
    mm = jnp.max(lg, axis=1, keepdims=True)
    e = jnp.exp(lg - mm)
    prob_ref[...] = e / jnp.sum(e, axis=1, keepdims=True)
    yhat_ref[0, 0] = jnp.where(lg[0, 1] > lg[0, 0], 1, 0).astype(jnp.int32)


def _tail(hg, W1, b1, Wi0, bi0, Wi1, bi1, Wcls, bcls, wsum, ssum, lab, iev):
    vm = lambda shape: pl.BlockSpec(shape, lambda: tuple(0 for _ in shape))
    sm = pl.BlockSpec(memory_space=pltpu.SMEM)
    return pl.pallas_call(
        _tail_body,
        in_specs=[
            vm((2 * K_SAMPLE, D_IN)), vm((D_IN, D1)), vm((1, D1)),
            vm((D1, 2)), vm((1, 2)), vm((D1, 2)), vm((1, 2)),
            vm((D1, 2)), vm((1, 2)), vm((1, D1)), sm, sm, sm,
        ],
        out_specs=[vm((1, 2)), vm((1, 2)), sm, sm],
        out_shape=[
            jax.ShapeDtypeStruct((1, 2), jnp.float32),
            jax.ShapeDtypeStruct((1, 2), jnp.float32),
            jax.ShapeDtypeStruct((1, 1), jnp.int32),
            jax.ShapeDtypeStruct((1, 1), jnp.float32),
        ],
    )(hg, W1, b1, Wi0, bi0, Wi1, bi1, Wcls, bcls, wsum, ssum, lab, iev)


def kernel(h, label, instance_eval, W1, b1, Wa, ba, Wb, bb, Wc, bc,
           Wcls, bcls, Wi0, bi0, Wi1, bi1):
    att_row, att_pad, wsum, ssum = _stream(
        h, W1, b1.reshape(1, D1), Wa, ba.reshape(1, D2),
        Wb, bb.reshape(1, D2), Wc, bc.reshape(1, 1))

    ids = _topk(att_pad)

    hg = _gather_rows(h, ids)

    lab = label.reshape(1, 1).astype(jnp.int32)
    iev = jnp.asarray(instance_eval, jnp.int32).reshape(1, 1)
    logits, prob, yhat, loss = _tail(
        hg, W1, b1.reshape(1, D1), Wi0, bi0.reshape(1, 2),
        Wi1, bi1.reshape(1, 2), Wcls, bcls.reshape(1, 2),
        wsum, ssum, lab, iev)

    return (logits, prob, yhat, att_row, loss.reshape(()))


# speedup vs baseline: 1.1562x; 1.0073x over previous
import jax
import jax.numpy as jnp
from jax.experimental import pallas as pl
from jax.experimental.pallas import tpu as pltpu
from jax.experimental.pallas import tpu_sc as plsc

N = 50000
D_IN = 1024
D1 = 512
D2 = 256
K_SAMPLE = 8
BLK = 2048
NBLK = -(-N // BLK)
ATT_W = NBLK * BLK
PAD_ROWS = ATT_W // 128
NEG_INF = float("-inf")
POS_INF = float("inf")


def _dot(x, y):
    xb = x if x.dtype == jnp.bfloat16 else x.astype(jnp.bfloat16)
    yb = y if y.dtype == jnp.bfloat16 else y.astype(jnp.bfloat16)
    return jax.lax.dot_general(
        xb, yb, (((x.ndim - 1,), (0,)), ((), ())),
        preferred_element_type=jnp.float32)


def _stream_body(h_ref, w1_ref, b1_ref, wa_ref, ba_ref, wb_ref, bb_ref,
                 wc_ref, bc_ref, att_ref, wsum_ref, ssum_ref, ids_ref,
                 scr_ref):
    i = pl.program_id(0)

    @pl.when(i == 0)
    def _():
        wsum_ref[...] = jnp.zeros_like(wsum_ref)
        ssum_ref[...] = jnp.zeros_like(ssum_ref)

    rowv = i * BLK + jax.lax.broadcasted_iota(jnp.int32, (BLK, 1), 0) < N
    feat = jnp.where(rowv,
                     jnp.maximum(_dot(h_ref[...], w1_ref[...]) + b1_ref[...],
                                 0.0), 0.0).astype(jnp.bfloat16)
    a = jnp.tanh(_dot(feat, wa_ref[...]) + ba_ref[...])
    b = jax.nn.sigmoid(_dot(feat, wb_ref[...]) + bb_ref[...])
    g = (a * b).astype(jnp.bfloat16)
    att = jax.lax.dot_general(
        wc_ref[...].astype(jnp.bfloat16), g, (((0,), (1,)), ((), ())),
        preferred_element_type=jnp.float32) + bc_ref[0, 0]
    att_ref[...] = att
    r16 = BLK // 128
    scr_ref[pl.ds(i * r16, r16), :] = att.reshape(r16, 128)
    lane = jax.lax.broadcasted_iota(jnp.int32, (1, BLK), 1)
    valid = i * BLK + lane < N
    sig = jnp.where(valid, jax.nn.sigmoid(att), 0.0)
    wsum_ref[...] += jax.lax.dot_general(
        sig.astype(jnp.bfloat16), feat, (((1,), (0,)), ((), ())),
        preferred_element_type=jnp.float32)
    ssum_ref[...] += jnp.sum(sig, axis=1, keepdims=True)

    @pl.when(i == NBLK - 1)
    def _():
        _topk_body(scr_ref, ids_ref)


def _stream(h, W1, b1, Wa, ba, Wb, bb, Wc, bc):
    return pl.pallas_call(
        _stream_body,
        grid=(NBLK,),
        in_specs=[
            pl.BlockSpec((BLK, D_IN), lambda i: (i, 0)),
            pl.BlockSpec((D_IN, D1), lambda i: (0, 0)),
            pl.BlockSpec((1, D1), lambda i: (0, 0)),
            pl.BlockSpec((D1, D2), lambda i: (0, 0)),
            pl.BlockSpec((1, D2), lambda i: (0, 0)),
            pl.BlockSpec((D1, D2), lambda i: (0, 0)),
            pl.BlockSpec((1, D2), lambda i: (0, 0)),
            pl.BlockSpec((D2, 1), lambda i: (0, 0)),
            pl.BlockSpec(memory_space=pltpu.SMEM),
        ],
        out_specs=[
            pl.BlockSpec((1, BLK), lambda i: (0, i)),
            pl.BlockSpec((1, D1), lambda i: (0, 0)),
            pl.BlockSpec((1, 1), lambda i: (0, 0)),
            pl.BlockSpec(memory_space=pltpu.SMEM),
        ],
        out_shape=[
            jax.ShapeDtypeStruct((1, N), jnp.float32),
            jax.ShapeDtypeStruct((1, D1), jnp.float32),
            jax.ShapeDtypeStruct((1, 1), jnp.float32),
            jax.ShapeDtypeStruct((1, 2 * K_SAMPLE), jnp.int32),
        ],
        scratch_shapes=[pltpu.VMEM((PAD_ROWS, 128), jnp.float32)],
        compiler_params=pltpu.CompilerParams(
            dimension_semantics=("arbitrary",)),
    )(h, W1, b1, Wa, ba, Wb, bb, Wc, bc)


def _tree(parts, op):
    while len(parts) > 1:
        nxt = [op(parts[j], parts[j + 1])
               for j in range(0, len(parts) - 1, 2)]
        if len(parts) % 2:
            nxt.append(parts[-1])
        parts = nxt
    return parts[0]


def _split(x):
    return [x[k * 8:(k + 1) * 8] for k in range(x.shape[0] // 8)]


def _colwise_top8(vals, rowi, lin, maximize):
    sentinel = NEG_INF if maximize else POS_INF
    red = jnp.maximum if maximize else jnp.minimum
    bigrow = jnp.int32(2**31 - 1)
    v = vals
    out_v, out_l = [], []
    for _ in range(K_SAMPLE):
        m = _tree(_split(v), red)
        m = _tree([m[j:j + 1] for j in range(8)], red)
        r = _tree(_split(jnp.where(v == m, rowi, bigrow)), jnp.minimum)
        r = _tree([r[j:j + 1] for j in range(8)], jnp.minimum)
        out_v.append(m)
        out_l.append(r * 128 + jax.lax.broadcasted_iota(
            jnp.int32, (1, 128), 1))
        v = jnp.where(rowi == r, sentinel, v)
    return jnp.concatenate(out_v, 0), jnp.concatenate(out_l, 0)


def _topk_body(att_ref, ids_ref):
    vals = att_ref[...]
    row = jax.lax.broadcasted_iota(jnp.int32, (PAD_ROWS, 128), 0)
    col = jax.lax.broadcasted_iota(jnp.int32, (PAD_ROWS, 128), 1)
    lin = row * 128 + col
    valid = lin < N
    big = jnp.int32(2**31 - 1)
    vt = jnp.where(valid, vals, NEG_INF)
    vb = jnp.where(valid, vals, POS_INF)
    tv, tl = _colwise_top8(vt, row, lin, True)
    bv, bl = _colwise_top8(vb, row, lin, False)
    for k in range(K_SAMPLE):
        mt = jnp.max(tv)
        it = jnp.min(jnp.where(tv == mt, tl, big))
        ids_ref[0, k] = it
        tv = jnp.where(tl == it, NEG_INF, tv)
        mb = jnp.min(bv)
        ib = jnp.min(jnp.where(bv == mb, bl, big))
        ids_ref[0, K_SAMPLE + k] = ib
        bv = jnp.where(bl == ib, POS_INF, bv)


def _topk(att_pad):
    return pl.pallas_call(
        _topk_body,
        in_specs=[pl.BlockSpec((PAD_ROWS, 128), lambda: (0, 0))],
        out_specs=pl.BlockSpec(memory_space=pltpu.SMEM),
        out_shape=jax.ShapeDtypeStruct((1, 2 * K_SAMPLE), jnp.int32),
    )(att_pad)


def _gather_rows(h, ids):
    mesh = plsc.VectorSubcoreMesh(core_axis_name="c", subcore_axis_name="s")

    @pl.kernel(out_type=jax.ShapeDtypeStruct((2 * K_SAMPLE, D_IN),
                                             jnp.float32),
               mesh=mesh,
               scratch_types=[pltpu.VMEM((1, 2 * K_SAMPLE), jnp.int32),
                              pltpu.VMEM((1, D_IN), jnp.float32),
                              pltpu.SemaphoreType.DMA])
    def kern(h_hbm, ids_hbm, o_hbm, ids_vmem, buf, sem):
        c = jax.lax.axis_index("c")
        s = jax.lax.axis_index("s")

        @pl.when(c == 0)
        def _():
            pltpu.async_copy(ids_hbm, ids_vmem, sem).wait()
            pltpu.sync_copy(h_hbm.at[ids_vmem.at[0, pl.ds(s, 1)]], buf)
            pltpu.async_copy(buf, o_hbm.at[pl.ds(s, 1)], sem).wait()

    return kern(h, ids)


def _tail_body(hg_ref, w1_ref, b1_ref, wi0_ref, bi0_ref, wi1_ref, bi1_ref,
               wcls_ref, bcls_ref, wsum_ref, ssum_ref, lab_ref, iev_ref,
               logits_ref, prob_ref, yhat_ref, loss_ref):
    fg = jnp.maximum(_dot(hg_ref[...], w1_ref[...]) + b1_ref[...], 0.0)

    def ce(lg):
        m = jnp.max(lg, axis=1, keepdims=True)
        lse = m + jnp.log(jnp.sum(jnp.exp(lg - m), axis=1, keepdims=True))
        rid = jax.lax.broadcasted_iota(jnp.int32, (2 * K_SAMPLE, 1), 0)
        ll = jnp.where(rid < K_SAMPLE, lg[:, 1:2], lg[:, 0:1])
        return jnp.sum(lse - ll) / (2.0 * K_SAMPLE)

    l0 = ce(_dot(fg, wi0_ref[...]) + bi0_ref[...])
    l1 = ce(_dot(fg, wi1_ref[...]) + bi1_ref[...])
    lab = lab_ref[0, 0]
    iev = iev_ref[0, 0]
    loss_ref[0, 0] = jnp.where(
        iev != 0, jnp.where(lab == 0, l0, l1), jnp.float32(0.0))

    bag = wsum_ref[...] / ssum_ref[0, 0]
    lg = _dot(bag, wcls_ref[...]) + bcls_ref[...]
    logits_ref[...] = lg
    mm = jnp.max(lg, axis=1, keepdims=True)
    e = jnp.exp(lg - mm)
    prob_ref[...] = e / jnp.sum(e, axis=1, keepdims=True)
    yhat_ref[0, 0] = jnp.where(lg[0, 1] > lg[0, 0], 1, 0).astype(jnp.int32)


def _tail(hg, W1, b1, Wi0, bi0, Wi1, bi1, Wcls, bcls, wsum, ssum, lab, iev):
    vm = lambda shape: pl.BlockSpec(shape, lambda: tuple(0 for _ in shape))
    sm = pl.BlockSpec(memory_space=pltpu.SMEM)
    return pl.pallas_call(
        _tail_body,
        in_specs=[
            vm((2 * K_SAMPLE, D_IN)), vm((D_IN, D1)), vm((1, D1)),
            vm((D1, 2)), vm((1, 2)), vm((D1, 2)), vm((1, 2)),
            vm((D1, 2)), vm((1, 2)), vm((1, D1)), sm, sm, sm,
        ],
        out_specs=[vm((1, 2)), vm((1, 2)), sm, sm],
        out_shape=[
            jax.ShapeDtypeStruct((1, 2), jnp.float32),
            jax.ShapeDtypeStruct((1, 2), jnp.float32),
            jax.ShapeDtypeStruct((1, 1), jnp.int32),
            jax.ShapeDtypeStruct((1, 1), jnp.float32),
        ],
    )(hg, W1, b1, Wi0, bi0, Wi1, bi1, Wcls, bcls, wsum, ssum, lab, iev)


def kernel(h, label, instance_eval, W1, b1, Wa, ba, Wb, bb, Wc, bc,
           Wcls, bcls, Wi0, bi0, Wi1, bi1):
    att_row, wsum, ssum, ids = _stream(
        h, W1, b1.reshape(1, D1), Wa, ba.reshape(1, D2),
        Wb, bb.reshape(1, D2), Wc, bc.reshape(1, 1))

    hg = _gather_rows(h, ids)

    lab = label.reshape(1, 1).astype(jnp.int32)
    iev = jnp.asarray(instance_eval, jnp.int32).reshape(1, 1)
    logits, prob, yhat, loss = _tail(
        hg, W1, b1.reshape(1, D1), Wi0, bi0.reshape(1, 2),
        Wi1, bi1.reshape(1, 2), Wcls, bcls.reshape(1, 2),
        wsum, ssum, lab, iev)

    return (logits, prob, yhat, att_row, loss.reshape(()))


# speedup vs baseline: 1.1571x; 1.0008x over previous
import jax
import jax.numpy as jnp
from jax.experimental import pallas as pl
from jax.experimental.pallas import tpu as pltpu
from jax.experimental.pallas import tpu_sc as plsc

N = 50000
D_IN = 1024
D1 = 512
D2 = 256
K_SAMPLE = 8
BLK = 2048
NBLK = -(-N // BLK)
ATT_W = NBLK * BLK
PAD_ROWS = ATT_W // 128
NEG_INF = float("-inf")
POS_INF = float("inf")


def _dot(x, y):
    xb = x if x.dtype == jnp.bfloat16 else x.astype(jnp.bfloat16)
    yb = y if y.dtype == jnp.bfloat16 else y.astype(jnp.bfloat16)
    return jax.lax.dot_general(
        xb, yb, (((x.ndim - 1,), (0,)), ((), ())),
        preferred_element_type=jnp.float32)


def _stream_body(h_ref, w1_ref, b1_ref, wa_ref, ba_ref, wb_ref, bb_ref,
                 wc_ref, bc_ref, att_ref, wsum_ref, ssum_ref, ids_ref,
                 scr_ref):
    i = pl.program_id(0)

    @pl.when(i == 0)
    def _():
        wsum_ref[...] = jnp.zeros_like(wsum_ref)
        ssum_ref[...] = jnp.zeros_like(ssum_ref)

    def block(masked):
        relu = jnp.maximum(_dot(h_ref[...], w1_ref[...]) + b1_ref[...], 0.0)
        if masked:
            rowv = (i * BLK
                    + jax.lax.broadcasted_iota(jnp.int32, (BLK, 1), 0)) < N
            relu = jnp.where(rowv, relu, 0.0)
        feat = relu.astype(jnp.bfloat16)
        a = jnp.tanh(_dot(feat, wa_ref[...]) + ba_ref[...])
        b = jax.nn.sigmoid(_dot(feat, wb_ref[...]) + bb_ref[...])
        g = (a * b).astype(jnp.bfloat16)
        att = jax.lax.dot_general(
            wc_ref[...].astype(jnp.bfloat16), g, (((0,), (1,)), ((), ())),
            preferred_element_type=jnp.float32) + bc_ref[0, 0]
        att_ref[...] = att
        r16 = BLK // 128
        scr_ref[pl.ds(i * r16, r16), :] = att.reshape(r16, 128)
        sig = jax.nn.sigmoid(att)
        if masked:
            lane = jax.lax.broadcasted_iota(jnp.int32, (1, BLK), 1)
            sig = jnp.where(i * BLK + lane < N, sig, 0.0)
        wsum_ref[...] += jax.lax.dot_general(
            sig.astype(jnp.bfloat16), feat, (((1,), (0,)), ((), ())),
            preferred_element_type=jnp.float32)
        ssum_ref[...] += jnp.sum(sig, axis=1, keepdims=True)

    @pl.when(i < NBLK - 1)
    def _():
        block(masked=False)

    @pl.when(i == NBLK - 1)
    def _():
        block(masked=True)
        _topk_body(scr_ref, ids_ref)


def _stream(h, W1, b1, Wa, ba, Wb, bb, Wc, bc):
    return pl.pallas_call(
        _stream_body,
        grid=(NBLK,),
        in_specs=[
            pl.BlockSpec((BLK, D_IN), lambda i: (i, 0)),
            pl.BlockSpec((D_IN, D1), lambda i: (0, 0)),
            pl.BlockSpec((1, D1), lambda i: (0, 0)),
            pl.BlockSpec((D1, D2), lambda i: (0, 0)),
            pl.BlockSpec((1, D2), lambda i: (0, 0)),
            pl.BlockSpec((D1, D2), lambda i: (0, 0)),
            pl.BlockSpec((1, D2), lambda i: (0, 0)),
            pl.BlockSpec((D2, 1), lambda i: (0, 0)),
            pl.BlockSpec(memory_space=pltpu.SMEM),
        ],
        out_specs=[
            pl.BlockSpec((1, BLK), lambda i: (0, i)),
            pl.BlockSpec((1, D1), lambda i: (0, 0)),
            pl.BlockSpec((1, 1), lambda i: (0, 0)),
            pl.BlockSpec(memory_space=pltpu.SMEM),
        ],
        out_shape=[
            jax.ShapeDtypeStruct((1, N), jnp.float32),
            jax.ShapeDtypeStruct((1, D1), jnp.float32),
            jax.ShapeDtypeStruct((1, 1), jnp.float32),
            jax.ShapeDtypeStruct((1, 2 * K_SAMPLE), jnp.int32),
        ],
        scratch_shapes=[pltpu.VMEM((PAD_ROWS, 128), jnp.float32)],
        compiler_params=pltpu.CompilerParams(
            dimension_semantics=("arbitrary",)),
    )(h, W1, b1, Wa, ba, Wb, bb, Wc, bc)


def _tree(parts, op):
    while len(parts) > 1:
        nxt = [op(parts[j], parts[j + 1])
               for j in range(0, len(parts) - 1, 2)]
        if len(parts) % 2:
            nxt.append(parts[-1])
        parts = nxt
    return parts[0]


def _split(x):
    return [x[k * 8:(k + 1) * 8] for k in range(x.shape[0] // 8)]


def _colwise_top8(vals, rowi, lin, maximize):
    sentinel = NEG_INF if maximize else POS_INF
    red = jnp.maximum if maximize else jnp.minimum
    bigrow = jnp.int32(2**31 - 1)
    v = vals
    out_v, out_l = [], []
    for _ in range(K_SAMPLE):
        m = _tree(_split(v), red)
        m = _tree([m[j:j + 1] for j in range(8)], red)
        r = _tree(_split(jnp.where(v == m, rowi, bigrow)), jnp.minimum)
        r = _tree([r[j:j + 1] for j in range(8)], jnp.minimum)
        out_v.append(m)
        out_l.append(r * 128 + jax.lax.broadcasted_iota(
            jnp.int32, (1, 128), 1))
        v = jnp.where(rowi == r, sentinel, v)
    return jnp.concatenate(out_v, 0), jnp.concatenate(out_l, 0)


def _topk_body(att_ref, ids_ref):
    vals = att_ref[...]
    row = jax.lax.broadcasted_iota(jnp.int32, (PAD_ROWS, 128), 0)
    col = jax.lax.broadcasted_iota(jnp.int32, (PAD_ROWS, 128), 1)
    lin = row * 128 + col
    valid = lin < N
    big = jnp.int32(2**31 - 1)
    vt = jnp.where(valid, vals, NEG_INF)
    vb = jnp.where(valid, vals, POS_INF)
    tv, tl = _colwise_top8(vt, row, lin, True)
    bv, bl = _colwise_top8(vb, row, lin, False)
    for k in range(K_SAMPLE):
        mt = jnp.max(tv)
        it = jnp.min(jnp.where(tv == mt, tl, big))
        ids_ref[0, k] = it
        tv = jnp.where(tl == it, NEG_INF, tv)
        mb = jnp.min(bv)
        ib = jnp.min(jnp.where(bv == mb, bl, big))
        ids_ref[0, K_SAMPLE + k] = ib
        bv = jnp.where(bl == ib, POS_INF, bv)


def _topk(att_pad):
    return pl.pallas_call(
        _topk_body,
        in_specs=[pl.BlockSpec((PAD_ROWS, 128), lambda: (0, 0))],
        out_specs=pl.BlockSpec(memory_space=pltpu.SMEM),
        out_shape=jax.ShapeDtypeStruct((1, 2 * K_SAMPLE), jnp.int32),
    )(att_pad)


def _gather_rows(h, ids):
    mesh = plsc.VectorSubcoreMesh(core_axis_name="c", subcore_axis_name="s")

    @pl.kernel(out_type=jax.ShapeDtypeStruct((2 * K_SAMPLE, D_IN),
                                             jnp.float32),
               mesh=mesh,
               scratch_types=[pltpu.VMEM((1, 2 * K_SAMPLE), jnp.int32),
                              pltpu.VMEM((1, D_IN), jnp.float32),
                              pltpu.SemaphoreType.DMA])
    def kern(h_hbm, ids_hbm, o_hbm, ids_vmem, buf, sem):
        c = jax.lax.axis_index("c")
        s = jax.lax.axis_index("s")

        @pl.when(c == 0)
        def _():
            pltpu.async_copy(ids_hbm, ids_vmem, sem).wait()
            pltpu.sync_copy(h_hbm.at[ids_vmem.at[0, pl.ds(s, 1)]], buf)
            pltpu.async_copy(buf, o_hbm.at[pl.ds(s, 1)], sem).wait()

    return kern(h, ids)


def _tail_body(hg_ref, w1_ref, b1_ref, wi0_ref, bi0_ref, wi1_ref, bi1_ref,
               wcls_ref, bcls_ref, wsum_ref, ssum_ref, lab_ref, iev_ref,
               logits_ref, prob_ref, yhat_ref, loss_ref):
    fg = jnp.maximum(_dot(hg_ref[...], w1_ref[...]) + b1_ref[...], 0.0)

    def ce(lg):
        m = jnp.max(lg, axis=1, keepdims=True)
        lse = m + jnp.log(jnp.sum(jnp.exp(lg - m), axis=1, keepdims=True))
        rid = jax.lax.broadcasted_iota(jnp.int32, (2 * K_SAMPLE, 1), 0)
        ll = jnp.where(rid < K_SAMPLE, lg[:, 1:2], lg[:, 0:1])
        return jnp.sum(lse - ll) / (2.0 * K_SAMPLE)

    l0 = ce(_dot(fg, wi0_ref[...]) + bi0_ref[...])
    l1 = ce(_dot(fg, wi1_ref[...]) + bi1_ref[...])
    lab = lab_ref[0, 0]
    iev = iev_ref[0, 0]
    loss_ref[0, 0] = jnp.where(
        iev != 0, jnp.where(lab == 0, l0, l1), jnp.float32(0.0))

    bag = wsum_ref[...] / ssum_ref[0, 0]
    lg = _dot(bag, wcls_ref[...]) + bcls_ref[...]
    logits_ref[...] = lg
    mm = jnp.max(lg, axis=1, keepdims=True)
    e = jnp.exp(lg - mm)
    prob_ref[...] = e / jnp.sum(e, axis=1, keepdims=True)
    yhat_ref[0, 0] = jnp.where(lg[0, 1] > lg[0, 0], 1, 0).astype(jnp.int32)


def _tail(hg, W1, b1, Wi0, bi0, Wi1, bi1, Wcls, bcls, wsum, ssum, lab, iev):
    vm = lambda shape: pl.BlockSpec(shape, lambda: tuple(0 for _ in shape))
    sm = pl.BlockSpec(memory_space=pltpu.SMEM)
    return pl.pallas_call(
        _tail_body,
        in_specs=[
            vm((2 * K_SAMPLE, D_IN)), vm((D_IN, D1)), vm((1, D1)),
            vm((D1, 2)), vm((1, 2)), vm((D1, 2)), vm((1, 2)),
            vm((D1, 2)), vm((1, 2)), vm((1, D1)), sm, sm, sm,
        ],
        out_specs=[vm((1, 2)), vm((1, 2)), sm, sm],
        out_shape=[
            jax.ShapeDtypeStruct((1, 2), jnp.float32),
            jax.ShapeDtypeStruct((1, 2), jnp.float32),
            jax.ShapeDtypeStruct((1, 1), jnp.int32),
            jax.ShapeDtypeStruct((1, 1), jnp.float32),
        ],
    )(hg, W1, b1, Wi0, bi0, Wi1, bi1, Wcls, bcls, wsum, ssum, lab, iev)


def kernel(h, label, instance_eval, W1, b1, Wa, ba, Wb, bb, Wc, bc,
           Wcls, bcls, Wi0, bi0, Wi1, bi1):
    att_row, wsum, ssum, ids = _stream(
        h, W1, b1.reshape(1, D1), Wa, ba.reshape(1, D2),
        Wb, bb.reshape(1, D2), Wc, bc.reshape(1, 1))

    hg = _gather_rows(h, ids)

    lab = label.reshape(1, 1).astype(jnp.int32)
    iev = jnp.asarray(instance_eval, jnp.int32).reshape(1, 1)
    logits, prob, yhat, loss = _tail(
        hg, W1, b1.reshape(1, D1), Wi0, bi0.reshape(1, 2),
        Wi1, bi1.reshape(1, 2), Wcls, bcls.reshape(1, 2),
        wsum, ssum, lab, iev)

    return (logits, prob, yhat, att_row, loss.reshape(()))


# speedup vs baseline: 1.1578x; 1.0005x over previous
import jax
import jax.numpy as jnp
from jax.experimental import pallas as pl
from jax.experimental.pallas import tpu as pltpu
from jax.experimental.pallas import tpu_sc as plsc

N = 50000
D_IN = 1024
D1 = 512
D2 = 256
K_SAMPLE = 8
BLK = 2048
NBLK = -(-N // BLK)
ATT_W = NBLK * BLK
PAD_ROWS = ATT_W // 128
NEG_INF = float("-inf")
POS_INF = float("inf")


def _dot(x, y):
    xb = x if x.dtype == jnp.bfloat16 else x.astype(jnp.bfloat16)
    yb = y if y.dtype == jnp.bfloat16 else y.astype(jnp.bfloat16)
    return jax.lax.dot_general(
        xb, yb, (((x.ndim - 1,), (0,)), ((), ())),
        preferred_element_type=jnp.float32)


def _stream_body(h_ref, w1_ref, b1_ref, wa_ref, ba_ref, wb_ref, bb_ref,
                 wc_ref, bc_ref, att_ref, wsum_ref, ssum_ref, ids_ref,
                 scr_ref):
    i = pl.program_id(0)

    @pl.when(i == 0)
    def _():
        wsum_ref[...] = jnp.zeros_like(wsum_ref)
        ssum_ref[...] = jnp.zeros_like(ssum_ref)

    rowv = i * BLK + jax.lax.broadcasted_iota(jnp.int32, (BLK, 1), 0) < N
    feat = jnp.where(rowv,
                     jnp.maximum(_dot(h_ref[...], w1_ref[...]) + b1_ref[...],
                                 0.0), 0.0).astype(jnp.bfloat16)
    a = jnp.tanh(_dot(feat, wa_ref[...]) + ba_ref[...])
    b = jax.nn.sigmoid(_dot(feat, wb_ref[...]) + bb_ref[...])
    g = (a * b).astype(jnp.bfloat16)
    att = jax.lax.dot_general(
        wc_ref[...].astype(jnp.bfloat16), g, (((0,), (1,)), ((), ())),
        preferred_element_type=jnp.float32) + bc_ref[0, 0]
    att_ref[...] = att
    r16 = BLK // 128
    scr_ref[pl.ds(i * r16, r16), :] = att.reshape(r16, 128)
    lane = jax.lax.broadcasted_iota(jnp.int32, (1, BLK), 1)
    valid = i * BLK + lane < N
    sig = jnp.where(valid, jax.nn.sigmoid(att), 0.0)
    wsum_ref[...] += jax.lax.dot_general(
        sig.astype(jnp.bfloat16), feat, (((1,), (0,)), ((), ())),
        preferred_element_type=jnp.float32)
    ssum_ref[...] += jnp.sum(sig, axis=1, keepdims=True)

    @pl.when(i == NBLK - 1)
    def _():
        _topk_body(scr_ref, ids_ref)


def _stream(h, W1, b1, Wa, ba, Wb, bb, Wc, bc):
    return pl.pallas_call(
        _stream_body,
        grid=(NBLK,),
        in_specs=[
            pl.BlockSpec((BLK, D_IN), lambda i: (i, 0)),
            pl.BlockSpec((D_IN, D1), lambda i: (0, 0)),
            pl.BlockSpec((1, D1), lambda i: (0, 0)),
            pl.BlockSpec((D1, D2), lambda i: (0, 0)),
            pl.BlockSpec((1, D2), lambda i: (0, 0)),
            pl.BlockSpec((D1, D2), lambda i: (0, 0)),
            pl.BlockSpec((1, D2), lambda i: (0, 0)),
            pl.BlockSpec((D2, 1), lambda i: (0, 0)),
            pl.BlockSpec(memory_space=pltpu.SMEM),
        ],
        out_specs=[
            pl.BlockSpec((1, BLK), lambda i: (0, i)),
            pl.BlockSpec((1, D1), lambda i: (0, 0)),
            pl.BlockSpec((1, 1), lambda i: (0, 0)),
            pl.BlockSpec(memory_space=pltpu.SMEM),
        ],
        out_shape=[
            jax.ShapeDtypeStruct((1, N), jnp.float32),
            jax.ShapeDtypeStruct((1, D1), jnp.float32),
            jax.ShapeDtypeStruct((1, 1), jnp.float32),
            jax.ShapeDtypeStruct((1, 2 * K_SAMPLE), jnp.int32),
        ],
        scratch_shapes=[pltpu.VMEM((PAD_ROWS, 128), jnp.float32)],
        compiler_params=pltpu.CompilerParams(
            dimension_semantics=("arbitrary",)),
    )(h, W1, b1, Wa, ba, Wb, bb, Wc, bc)


def _tree(parts, op):
    while len(parts) > 1:
        nxt = [op(parts[j], parts[j + 1])
               for j in range(0, len(parts) - 1, 2)]
        if len(parts) % 2:
            nxt.append(parts[-1])
        parts = nxt
    return parts[0]


def _split(x):
    return [x[k * 8:(k + 1) * 8] for k in range(x.shape[0] // 8)]


def _colwise_top8(vals, rowi, lin, maximize):
    sentinel = NEG_INF if maximize else POS_INF
    red = jnp.maximum if maximize else jnp.minimum
    bigrow = jnp.int32(2**31 - 1)
    v = vals
    out_v, out_l = [], []
    for _ in range(K_SAMPLE):
        m = _tree(_split(v), red)
        m = _tree([m[j:j + 1] for j in range(8)], red)
        r = _tree(_split(jnp.where(v == m, rowi, bigrow)), jnp.minimum)
        r = _tree([r[j:j + 1] for j in range(8)], jnp.minimum)
        out_v.append(m)
        out_l.append(r * 128 + jax.lax.broadcasted_iota(
            jnp.int32, (1, 128), 1))
        v = jnp.where(rowi == r, sentinel, v)
    return jnp.concatenate(out_v, 0), jnp.concatenate(out_l, 0)


def _topk_body(att_ref, ids_ref):
    vals = att_ref[...]
    row = jax.lax.broadcasted_iota(jnp.int32, (PAD_ROWS, 128), 0)
    col = jax.lax.broadcasted_iota(jnp.int32, (PAD_ROWS, 128), 1)
    lin = row * 128 + col
    valid = lin < N
    big = jnp.int32(2**31 - 1)
    vt = jnp.where(valid, vals, NEG_INF)
    vb = jnp.where(valid, vals, POS_INF)
    tv, tl = _colwise_top8(vt, row, lin, True)
    bv, bl = _colwise_top8(vb, row, lin, False)
    for k in range(K_SAMPLE):
        mt = jnp.max(tv)
        it = jnp.min(jnp.where(tv == mt, tl, big))
        ids_ref[0, k] = it
        tv = jnp.where(tl == it, NEG_INF, tv)
        mb = jnp.min(bv)
        ib = jnp.min(jnp.where(bv == mb, bl, big))
        ids_ref[0, K_SAMPLE + k] = ib
        bv = jnp.where(bl == ib, POS_INF, bv)


def _topk(att_pad):
    return pl.pallas_call(
        _topk_body,
        in_specs=[pl.BlockSpec((PAD_ROWS, 128), lambda: (0, 0))],
        out_specs=pl.BlockSpec(memory_space=pltpu.SMEM),
        out_shape=jax.ShapeDtypeStruct((1, 2 * K_SAMPLE), jnp.int32),
    )(att_pad)


def _gather_rows(h, ids):
    mesh = plsc.VectorSubcoreMesh(core_axis_name="c", subcore_axis_name="s")

    @pl.kernel(out_type=jax.ShapeDtypeStruct((2 * K_SAMPLE, D_IN),
                                             jnp.float32),
               mesh=mesh,
               scratch_types=[pltpu.VMEM((1, 2 * K_SAMPLE), jnp.int32),
                              pltpu.VMEM((1, D_IN), jnp.float32),
                              pltpu.SemaphoreType.DMA])
    def kern(h_hbm, ids_hbm, o_hbm, ids_vmem, buf, sem):
        c = jax.lax.axis_index("c")
        s = jax.lax.axis_index("s")

        @pl.when(c == 0)
        def _():
            pltpu.async_copy(ids_hbm, ids_vmem, sem).wait()
            pltpu.sync_copy(h_hbm.at[ids_vmem.at[0, pl.ds(s, 1)]], buf)
            pltpu.async_copy(buf, o_hbm.at[pl.ds(s, 1)], sem).wait()

    return kern(h, ids)


def _tail_body(hg_ref, w1_ref, b1_ref, wi0_ref, bi0_ref, wi1_ref, bi1_ref,
               wcls_ref, bcls_ref, wsum_ref, ssum_ref, lab_ref, iev_ref,
               logits_ref, prob_ref, yhat_ref, loss_ref):
    fg = jnp.maximum(_dot(hg_ref[...], w1_ref[...]) + b1_ref[...], 0.0)

    def ce(lg):
        m = jnp.max(lg, axis=1, keepdims=True)
        lse = m + jnp.log(jnp.sum(jnp.exp(lg - m), axis=1, keepdims=True))
        rid = jax.lax.broadcasted_iota(jnp.int32, (2 * K_SAMPLE, 1), 0)
        ll = jnp.where(rid < K_SAMPLE, lg[:, 1:2], lg[:, 0:1])
        return jnp.sum(lse - ll) / (2.0 * K_SAMPLE)

    l0 = ce(_dot(fg, wi0_ref[...]) + bi0_ref[...])
    l1 = ce(_dot(fg, wi1_ref[...]) + bi1_ref[...])
    lab = lab_ref[0, 0]
    iev = iev_ref[0, 0]
    loss_ref[0, 0] = jnp.where(
        iev != 0, jnp.where(lab == 0, l0, l1), jnp.float32(0.0))

    bag = wsum_ref[...] / ssum_ref[0, 0]
    lg = _dot(bag, wcls_ref[...]) + bcls_ref[...]
    logits_ref[...] = lg
    mm = jnp.max(lg, axis=1, keepdims=True)
    e = jnp.exp(lg - mm)
    prob_ref[...] = e / jnp.sum(e, axis=1, keepdims=True)
    yhat_ref[0, 0] = jnp.where(lg[0, 1] > lg[0, 0], 1, 0).astype(jnp.int32)


def _tail(hg, W1, b1, Wi0, bi0, Wi1, bi1, Wcls, bcls, wsum, ssum, lab, iev):
    vm = lambda shape: pl.BlockSpec(shape, lambda: tuple(0 for _ in shape))
    sm = pl.BlockSpec(memory_space=pltpu.SMEM)
    return pl.pallas_call(
        _tail_body,
        in_specs=[
            vm((2 * K_SAMPLE, D_IN)), vm((D_IN, D1)), vm((1, D1)),
            vm((D1, 2)), vm((1, 2)), vm((D1, 2)), vm((1, 2)),
            vm((D1, 2)), vm((1, 2)), vm((1, D1)), sm, sm, sm,
        ],
        out_specs=[vm((1, 2)), vm((1, 2)), sm, sm],
        out_shape=[
            jax.ShapeDtypeStruct((1, 2), jnp.float32),
            jax.ShapeDtypeStruct((1, 2), jnp.float32),
            jax.ShapeDtypeStruct((1, 1), jnp.int32),
            jax.ShapeDtypeStruct((1, 1), jnp.float32),
        ],
    )(hg, W1, b1, Wi0, bi0, Wi1, bi1, Wcls, bcls, wsum, ssum, lab, iev)


def kernel(h, label, instance_eval, W1, b1, Wa, ba, Wb, bb, Wc, bc,
           Wcls, bcls, Wi0, bi0, Wi1, bi1):
    att_row, wsum, ssum, ids = _stream(
        h, W1, b1.reshape(1, D1), Wa, ba.reshape(1, D2),
        Wb, bb.reshape(1, D2), Wc, bc.reshape(1, 1))

    hg = _gather_rows(h, ids)

    lab = label.reshape(1, 1).astype(jnp.int32)
    iev = jnp.asarray(instance_eval, jnp.int32).reshape(1, 1)
    logits, prob, yhat, loss = _tail(
        hg, W1, b1.reshape(1, D1), Wi0, bi0.reshape(1, 2),
        Wi1, bi1.reshape(1, 2), Wcls, bcls.reshape(1, 2),
        wsum, ssum, lab, iev)

    return (logits, prob, yhat, att_row, loss.reshape(()))
